# baseline (speedup 1.0000x reference)
import functools

import jax
import jax.numpy as jnp
from jax import lax
from jax.experimental import pallas as pl
from jax.experimental.pallas import tpu as pltpu
from jax.experimental.pallas import tpu_sc as plsc

B = 1024
IN_DIM = 768
T = 4
E_DIM = 256
K = 8192
N_LAYERS = 4
BETA = 0.65
ALIGN_DIM = 512
R = B * T

NC = 2
NS = 16
NW = NC * NS
RPW = R // NW

RB = 256
KB = 1024
RT = R // RB
KT = K // KB


def _enc_body(x_ref, w_ref, b_ref, o_ref):
    o_ref[...] = (
        jnp.dot(x_ref[...], w_ref[...], preferred_element_type=jnp.float32)
        + b_ref[...]
    )


def _encode(x, w, b):
    return pl.pallas_call(
        _enc_body,
        out_shape=jax.ShapeDtypeStruct((B, T * E_DIM), jnp.float32),
    )(x, w, b.reshape(1, T * E_DIM))


U = 2
RTG = RT // U


def _dist_body(res_ref, cb_ref, idx_ref, cb2_ref, c2_ref, r2_ref,
               min_ref, cid_ref):
    kt = pl.program_id(0)
    rtg = pl.program_id(1)
    ones = jnp.ones((1, E_DIM), jnp.float32)
    C = KB // 8

    @pl.when(rtg == 0)
    def _():
        cb = cb_ref[...]
        cb2_ref[...] = cb + cb
        c2_ref[...] = lax.dot_general(
            cb * cb, ones, (((1,), (1,)), ((), ())),
            preferred_element_type=jnp.float32,
        )

    @pl.when(kt == 0)
    def _():
        res = res_ref[...]
        r2_ref[:, pl.ds(rtg * U * RB, U * RB)] = lax.dot_general(
            ones, res * res, (((1,), (1,)), ((), ())),
            preferred_element_type=jnp.float32,
        )

    cb2 = cb2_ref[...]
    c2 = c2_ref[...]
    first = kt == 0
    for u in range(U):
        res_u = res_ref[u * RB:(u + 1) * RB, :]
        r2_u = r2_ref[:, pl.ds((rtg * U + u) * RB, RB)]
        dots2 = lax.dot_general(
            cb2, res_u, (((1,), (1,)), ((), ())),
            preferred_element_type=jnp.float32,
        )
        sl = pl.ds((rtg * U + u) * RB, RB)
        om = jnp.where(first, jnp.float32(jnp.inf), min_ref[:, sl])
        oi = jnp.where(first, jnp.float32(0), cid_ref[:, sl])
        r2b = jnp.broadcast_to(r2_u, (8, RB))
        for c in range(C):
            d = (r2b - dots2[c * 8:(c + 1) * 8, :]) + c2[c * 8:(c + 1) * 8]
            idc = (kt * C + c).astype(jnp.float32)
            better = d < om
            oi = jnp.where(better, idc, oi)
            om = jnp.where(better, d, om)
        min_ref[:, sl] = om
        cid_ref[:, sl] = oi

        ki = (oi.astype(jnp.int32) * 8
              + lax.broadcasted_iota(jnp.int32, (8, RB), 0))
        gm = jnp.min(om, axis=0, keepdims=True)
        idx_ref[0, 0, :, u * RB:(u + 1) * RB] = jnp.min(
            jnp.where(om == gm, ki, jnp.int32(2**30)), axis=0, keepdims=True
        )


def _nearest(res, cb):
    idx3 = pl.pallas_call(
        _dist_body,
        grid=(KT, RTG),
        in_specs=[
            pl.BlockSpec((U * RB, E_DIM), lambda kt, rtg: (rtg, 0)),
            pl.BlockSpec((KB, E_DIM), lambda kt, rtg: (kt, 0)),
        ],
        out_specs=pl.BlockSpec(
            (1, 1, 1, U * RB), lambda kt, rtg: (kt, rtg, 0, 0)),
        out_shape=jax.ShapeDtypeStruct((KT, RTG, 1, U * RB), jnp.int32),
        scratch_shapes=[
            pltpu.VMEM((KB, E_DIM), jnp.float32),
            pltpu.VMEM((KB, 1), jnp.float32),
            pltpu.VMEM((1, R), jnp.float32),
            pltpu.VMEM((8, R), jnp.float32),
            pltpu.VMEM((8, R), jnp.float32),
        ],
    )(res, cb)
    return idx3[KT - 1].reshape(R)


def _sc_update_body(cb_hbm, idx_hbm, res_hbm, res_out, sq_out,
                    idx_v, q_v, r_v, sq_v, sem):
    wid = lax.axis_index("s") * NC + lax.axis_index("c")
    base = wid * RPW
    pltpu.sync_copy(idx_hbm.at[pl.ds(base, RPW)], idx_v)
    pltpu.async_copy(cb_hbm.at[idx_v], q_v, sem).wait()
    pltpu.sync_copy(res_hbm.at[pl.ds(base, RPW)], r_v)

    def row(i, acc):
        for j in range(E_DIM // 16):
            sl = pl.ds(j * 16, 16)
            r = r_v[i, sl]
            q = q_v[i, sl]
            t = q - r
            q_st = r + t
            r_v[i, sl] = r - q_st
            acc = acc + t * t
        return acc

    acc = lax.fori_loop(0, RPW, row, jnp.zeros((16,), jnp.float32))
    sq_v[...] = acc
    pltpu.sync_copy(r_v, res_out.at[pl.ds(base, RPW)])
    pltpu.sync_copy(sq_v, sq_out.at[wid])


@functools.lru_cache(maxsize=1)
def _build_sc_update():
    return pl.kernel(
        _sc_update_body,
        out_type=(
            jax.ShapeDtypeStruct((R, E_DIM), jnp.float32),
            jax.ShapeDtypeStruct((NW, 16), jnp.float32),
        ),
        mesh=plsc.VectorSubcoreMesh(core_axis_name="c", subcore_axis_name="s",
                                    num_cores=NC, num_subcores=NS),
        scratch_types=[
            pltpu.VMEM((RPW,), jnp.int32),
            pltpu.VMEM((RPW, E_DIM), jnp.float32),
            pltpu.VMEM((RPW, E_DIM), jnp.float32),
            pltpu.VMEM((16,), jnp.float32),
            pltpu.SemaphoreType.DMA,
        ],
    )


DB = 256
DT = B // DB


def _dec_body(res_ref, xe_ref, wdt_ref, bdt_ref, wd_ref, bd_ref,
              wa_ref, ba_ref, sq_ref, xd_ref, rec_ref, al_ref, loss_ref):
    t = pl.program_id(0)
    q = xe_ref[...] - res_ref[...]
    wdt = wdt_ref[...]
    bdt = bdt_ref[...]
    xd_parts = []
    al_parts = []
    for tok in range(T):
        qt = q[:, tok * E_DIM:(tok + 1) * E_DIM]
        xdt = jnp.dot(qt, wdt, preferred_element_type=jnp.float32) + bdt
        xd_parts.append(xdt)
        al_parts.append(
            jnp.dot(xdt, wa_ref[...], preferred_element_type=jnp.float32)
            + ba_ref[...]
        )
    xd = jnp.concatenate(xd_parts, axis=1)
    xd_ref[...] = xd
    rec_ref[...] = (
        jnp.dot(xd, wd_ref[...], preferred_element_type=jnp.float32)
        + bd_ref[...]
    )
    al_ref[...] = jnp.concatenate(al_parts, axis=1)

    @pl.when(t == 0)
    def _():
        loss_ref[0, 0] = (
            (1.0 + BETA) * jnp.sum(sq_ref[...]) / jnp.float32(R * E_DIM)
        )


def _decode(res4, xe, wdt, bdt, wd, bd, wa, ba, sq):
    return pl.pallas_call(
        _dec_body,
        grid=(DT,),
        in_specs=[
            pl.BlockSpec((DB, T * E_DIM), lambda t: (t, 0)),
            pl.BlockSpec((DB, T * E_DIM), lambda t: (t, 0)),
            pl.BlockSpec((E_DIM, E_DIM), lambda t: (0, 0)),
            pl.BlockSpec((1, E_DIM), lambda t: (0, 0)),
            pl.BlockSpec((T * E_DIM, IN_DIM), lambda t: (0, 0)),
            pl.BlockSpec((1, IN_DIM), lambda t: (0, 0)),
            pl.BlockSpec((E_DIM, ALIGN_DIM), lambda t: (0, 0)),
            pl.BlockSpec((1, ALIGN_DIM), lambda t: (0, 0)),
            pl.BlockSpec((N_LAYERS * NW, 16), lambda t: (0, 0)),
        ],
        out_specs=[
            pl.BlockSpec((DB, T * E_DIM), lambda t: (t, 0)),
            pl.BlockSpec((DB, IN_DIM), lambda t: (t, 0)),
            pl.BlockSpec((DB, T * ALIGN_DIM), lambda t: (t, 0)),
            pl.BlockSpec(memory_space=pltpu.SMEM),
        ],
        out_shape=[
            jax.ShapeDtypeStruct((B, T * E_DIM), jnp.float32),
            jax.ShapeDtypeStruct((B, IN_DIM), jnp.float32),
            jax.ShapeDtypeStruct((B, T * ALIGN_DIM), jnp.float32),
            jax.ShapeDtypeStruct((1, 1), jnp.float32),
        ],
    )(res4, xe, wdt, bdt.reshape(1, E_DIM), wd, bd.reshape(1, IN_DIM),
      wa, ba.reshape(1, ALIGN_DIM), sq)


def kernel(video_patches, W_enc, b_enc, cb0, cb1, cb2, cb3,
           W_dec_tok, b_dec_tok, W_dec, b_dec, W_align, b_align):
    xe = _encode(video_patches, W_enc, b_enc)
    res = xe.reshape(R, E_DIM)
    idxs = []
    sqs = []
    for cb in (cb0, cb1, cb2, cb3):
        idx = _nearest(res, cb)
        res, sq = _build_sc_update()(cb, idx, res)
        idxs.append(idx)
        sqs.append(sq)
    sq_all = jnp.concatenate(sqs, axis=0)
    xd, rec, al, loss = _decode(
        res.reshape(B, T * E_DIM), xe, W_dec_tok, b_dec_tok,
        W_dec, b_dec, W_align, b_align, sq_all)
    indices = jnp.stack(idxs, axis=-1).reshape(B, T, N_LAYERS)
    return (
        rec,
        loss.reshape(()),
        indices,
        xe.reshape(B, T, E_DIM),
        xd.reshape(B, T, E_DIM),
        al.reshape(B, T, ALIGN_DIM),
    )


# speedup vs baseline: 1.3661x; 1.3661x over previous
import functools

import jax
import jax.numpy as jnp
from jax import lax
from jax.experimental import pallas as pl
from jax.experimental.pallas import tpu as pltpu
from jax.experimental.pallas import tpu_sc as plsc

B = 1024
IN_DIM = 768
T = 4
E_DIM = 256
K = 8192
N_LAYERS = 4
BETA = 0.65
ALIGN_DIM = 512
R = B * T

NC = 2
NS = 16
NW = NC * NS
RPW = R // NW

RB = 256
KB = 1024
RT = R // RB
KT = K // KB


def _enc_body(x_ref, w_ref, b_ref, o_ref):
    o_ref[...] = (
        jnp.dot(x_ref[...], w_ref[...], preferred_element_type=jnp.float32)
        + b_ref[...]
    )


def _encode(x, w, b):
    return pl.pallas_call(
        _enc_body,
        out_shape=jax.ShapeDtypeStruct((B, T * E_DIM), jnp.float32),
    )(x, w, b.reshape(1, T * E_DIM))


U = 8
RTG = RT // U


def _dist_body(res_ref, cb_ref, idx_ref, cb2_ref, c2_ref, r2_ref,
               min_ref, cid_ref):
    kt = pl.program_id(0)
    rtg = pl.program_id(1)
    ones = jnp.ones((1, E_DIM), jnp.float32)
    C = KB // 8

    @pl.when(rtg == 0)
    def _():
        cb = cb_ref[...]
        cb2_ref[...] = cb + cb
        c2_ref[...] = lax.dot_general(
            cb * cb, ones, (((1,), (1,)), ((), ())),
            preferred_element_type=jnp.float32,
        )

    @pl.when(kt == 0)
    def _():
        res = res_ref[...]
        r2_ref[:, pl.ds(rtg * U * RB, U * RB)] = lax.dot_general(
            ones, res * res, (((1,), (1,)), ((), ())),
            preferred_element_type=jnp.float32,
        )

    cb2 = cb2_ref[...]
    c2 = c2_ref[...]
    first = kt == 0
    for u in range(U):
        res_u = res_ref[u * RB:(u + 1) * RB, :]
        r2_u = r2_ref[:, pl.ds((rtg * U + u) * RB, RB)]
        dots2 = lax.dot_general(
            cb2, res_u, (((1,), (1,)), ((), ())),
            preferred_element_type=jnp.float32,
        )
        sl = pl.ds((rtg * U + u) * RB, RB)
        om = jnp.where(first, jnp.float32(jnp.inf), min_ref[:, sl])
        oi = jnp.where(first, jnp.float32(0), cid_ref[:, sl])
        r2b = jnp.broadcast_to(r2_u, (8, RB))
        for c in range(C):
            d = (r2b - dots2[c * 8:(c + 1) * 8, :]) + c2[c * 8:(c + 1) * 8]
            idc = (kt * C + c).astype(jnp.float32)
            better = d < om
            oi = jnp.where(better, idc, oi)
            om = jnp.where(better, d, om)
        min_ref[:, sl] = om
        cid_ref[:, sl] = oi

        ki = (oi.astype(jnp.int32) * 8
              + lax.broadcasted_iota(jnp.int32, (8, RB), 0))
        gm = jnp.min(om, axis=0, keepdims=True)
        idx_ref[0, 0, :, u * RB:(u + 1) * RB] = jnp.min(
            jnp.where(om == gm, ki, jnp.int32(2**30)), axis=0, keepdims=True
        )


def _nearest(res, cb):
    idx3 = pl.pallas_call(
        _dist_body,
        grid=(KT, RTG),
        in_specs=[
            pl.BlockSpec((U * RB, E_DIM), lambda kt, rtg: (rtg, 0)),
            pl.BlockSpec((KB, E_DIM), lambda kt, rtg: (kt, 0)),
        ],
        out_specs=pl.BlockSpec(
            (1, 1, 1, U * RB), lambda kt, rtg: (kt, rtg, 0, 0)),
        out_shape=jax.ShapeDtypeStruct((KT, RTG, 1, U * RB), jnp.int32),
        scratch_shapes=[
            pltpu.VMEM((KB, E_DIM), jnp.float32),
            pltpu.VMEM((KB, 1), jnp.float32),
            pltpu.VMEM((1, R), jnp.float32),
            pltpu.VMEM((8, R), jnp.float32),
            pltpu.VMEM((8, R), jnp.float32),
        ],
    )(res, cb)
    return idx3[KT - 1].reshape(R)


def _sc_update_body(cb_hbm, idx_hbm, res_hbm, res_out, sq_out,
                    idx_v, q_v, r_v, sq_v, sem):
    wid = lax.axis_index("s") * NC + lax.axis_index("c")
    base = wid * RPW
    pltpu.sync_copy(idx_hbm.at[pl.ds(base, RPW)], idx_v)
    pltpu.async_copy(cb_hbm.at[idx_v], q_v, sem).wait()
    pltpu.sync_copy(res_hbm.at[pl.ds(base, RPW)], r_v)

    def row(i, acc):
        for j in range(E_DIM // 16):
            sl = pl.ds(j * 16, 16)
            r = r_v[i, sl]
            q = q_v[i, sl]
            t = q - r
            q_st = r + t
            r_v[i, sl] = r - q_st
            acc = acc + t * t
        return acc

    acc = lax.fori_loop(0, RPW, row, jnp.zeros((16,), jnp.float32))
    sq_v[...] = acc
    pltpu.sync_copy(r_v, res_out.at[pl.ds(base, RPW)])
    pltpu.sync_copy(sq_v, sq_out.at[wid])


@functools.lru_cache(maxsize=1)
def _build_sc_update():
    return pl.kernel(
        _sc_update_body,
        out_type=(
            jax.ShapeDtypeStruct((R, E_DIM), jnp.float32),
            jax.ShapeDtypeStruct((NW, 16), jnp.float32),
        ),
        mesh=plsc.VectorSubcoreMesh(core_axis_name="c", subcore_axis_name="s",
                                    num_cores=NC, num_subcores=NS),
        scratch_types=[
            pltpu.VMEM((RPW,), jnp.int32),
            pltpu.VMEM((RPW, E_DIM), jnp.float32),
            pltpu.VMEM((RPW, E_DIM), jnp.float32),
            pltpu.VMEM((16,), jnp.float32),
            pltpu.SemaphoreType.DMA,
        ],
    )


DB = 256
DT = B // DB


def _dec_body(res_ref, xe_ref, wdt_ref, bdt_ref, wd_ref, bd_ref,
              wa_ref, ba_ref, sq_ref, xd_ref, rec_ref, al_ref, loss_ref):
    t = pl.program_id(0)
    q = xe_ref[...] - res_ref[...]
    wdt = wdt_ref[...]
    bdt = bdt_ref[...]
    xd_parts = []
    al_parts = []
    for tok in range(T):
        qt = q[:, tok * E_DIM:(tok + 1) * E_DIM]
        xdt = jnp.dot(qt, wdt, preferred_element_type=jnp.float32) + bdt
        xd_parts.append(xdt)
        al_parts.append(
            jnp.dot(xdt, wa_ref[...], preferred_element_type=jnp.float32)
            + ba_ref[...]
        )
    xd = jnp.concatenate(xd_parts, axis=1)
    xd_ref[...] = xd
    rec_ref[...] = (
        jnp.dot(xd, wd_ref[...], preferred_element_type=jnp.float32)
        + bd_ref[...]
    )
    al_ref[...] = jnp.concatenate(al_parts, axis=1)

    @pl.when(t == 0)
    def _():
        loss_ref[0, 0] = (
            (1.0 + BETA) * jnp.sum(sq_ref[...]) / jnp.float32(R * E_DIM)
        )


def _decode(res4, xe, wdt, bdt, wd, bd, wa, ba, sq):
    return pl.pallas_call(
        _dec_body,
        grid=(DT,),
        in_specs=[
            pl.BlockSpec((DB, T * E_DIM), lambda t: (t, 0)),
            pl.BlockSpec((DB, T * E_DIM), lambda t: (t, 0)),
            pl.BlockSpec((E_DIM, E_DIM), lambda t: (0, 0)),
            pl.BlockSpec((1, E_DIM), lambda t: (0, 0)),
            pl.BlockSpec((T * E_DIM, IN_DIM), lambda t: (0, 0)),
            pl.BlockSpec((1, IN_DIM), lambda t: (0, 0)),
            pl.BlockSpec((E_DIM, ALIGN_DIM), lambda t: (0, 0)),
            pl.BlockSpec((1, ALIGN_DIM), lambda t: (0, 0)),
            pl.BlockSpec((N_LAYERS * NW, 16), lambda t: (0, 0)),
        ],
        out_specs=[
            pl.BlockSpec((DB, T * E_DIM), lambda t: (t, 0)),
            pl.BlockSpec((DB, IN_DIM), lambda t: (t, 0)),
            pl.BlockSpec((DB, T * ALIGN_DIM), lambda t: (t, 0)),
            pl.BlockSpec(memory_space=pltpu.SMEM),
        ],
        out_shape=[
            jax.ShapeDtypeStruct((B, T * E_DIM), jnp.float32),
            jax.ShapeDtypeStruct((B, IN_DIM), jnp.float32),
            jax.ShapeDtypeStruct((B, T * ALIGN_DIM), jnp.float32),
            jax.ShapeDtypeStruct((1, 1), jnp.float32),
        ],
    )(res4, xe, wdt, bdt.reshape(1, E_DIM), wd, bd.reshape(1, IN_DIM),
      wa, ba.reshape(1, ALIGN_DIM), sq)


def kernel(video_patches, W_enc, b_enc, cb0, cb1, cb2, cb3,
           W_dec_tok, b_dec_tok, W_dec, b_dec, W_align, b_align):
    xe = _encode(video_patches, W_enc, b_enc)
    res = xe.reshape(R, E_DIM)
    idxs = []
    sqs = []
    for cb in (cb0, cb1, cb2, cb3):
        idx = _nearest(res, cb)
        res, sq = _build_sc_update()(cb, idx, res)
        idxs.append(idx)
        sqs.append(sq)
    sq_all = jnp.concatenate(sqs, axis=0)
    xd, rec, al, loss = _decode(
        res.reshape(B, T * E_DIM), xe, W_dec_tok, b_dec_tok,
        W_dec, b_dec, W_align, b_align, sq_all)
    indices = jnp.stack(idxs, axis=-1).reshape(B, T, N_LAYERS)
    return (
        rec,
        loss.reshape(()),
        indices,
        xe.reshape(B, T, E_DIM),
        xd.reshape(B, T, E_DIM),
        al.reshape(B, T, ALIGN_DIM),
    )


# speedup vs baseline: 1.4036x; 1.0275x over previous
import functools

import jax
import jax.numpy as jnp
from jax import lax
from jax.experimental import pallas as pl
from jax.experimental.pallas import tpu as pltpu
from jax.experimental.pallas import tpu_sc as plsc

B = 1024
IN_DIM = 768
T = 4
E_DIM = 256
K = 8192
N_LAYERS = 4
BETA = 0.65
ALIGN_DIM = 512
R = B * T

NC = 2
NS = 16
NW = NC * NS
RPW = R // NW

RB = 256
KB = 1024
RT = R // RB
KT = K // KB


def _enc_body(x_ref, w_ref, b_ref, o_ref):
    o_ref[...] = (
        jnp.dot(x_ref[...], w_ref[...], preferred_element_type=jnp.float32)
        + b_ref[...]
    )


def _encode(x, w, b):
    return pl.pallas_call(
        _enc_body,
        out_shape=jax.ShapeDtypeStruct((B, T * E_DIM), jnp.float32),
    )(x, w, b.reshape(1, T * E_DIM))


U = 16
RTG = RT // U


def _dist_body(res_ref, cb_ref, idx_ref, cb2_ref, c2_ref, r2_ref,
               min_ref, cid_ref):
    kt = pl.program_id(0)
    rtg = pl.program_id(1)
    ones = jnp.ones((1, E_DIM), jnp.float32)
    C = KB // 8

    @pl.when(rtg == 0)
    def _():
        cb = cb_ref[...]
        cb2_ref[...] = cb + cb
        c2_ref[...] = lax.dot_general(
            cb * cb, ones, (((1,), (1,)), ((), ())),
            preferred_element_type=jnp.float32,
        )

    @pl.when(kt == 0)
    def _():
        res = res_ref[...]
        r2_ref[:, pl.ds(rtg * U * RB, U * RB)] = lax.dot_general(
            ones, res * res, (((1,), (1,)), ((), ())),
            preferred_element_type=jnp.float32,
        )

    cb2 = cb2_ref[...]
    c2 = c2_ref[...]
    first = kt == 0
    for u in range(U):
        res_u = res_ref[u * RB:(u + 1) * RB, :]
        r2_u = r2_ref[:, pl.ds((rtg * U + u) * RB, RB)]
        dots2 = lax.dot_general(
            cb2, res_u, (((1,), (1,)), ((), ())),
            preferred_element_type=jnp.float32,
        )
        sl = pl.ds((rtg * U + u) * RB, RB)
        om = jnp.where(first, jnp.float32(jnp.inf), min_ref[:, sl])
        oi = jnp.where(first, jnp.float32(0), cid_ref[:, sl])
        r2b = jnp.broadcast_to(r2_u, (8, RB))
        for c in range(C):
            d = (r2b - dots2[c * 8:(c + 1) * 8, :]) + c2[c * 8:(c + 1) * 8]
            idc = (kt * C + c).astype(jnp.float32)
            better = d < om
            oi = jnp.where(better, idc, oi)
            om = jnp.where(better, d, om)
        min_ref[:, sl] = om
        cid_ref[:, sl] = oi

        ki = (oi.astype(jnp.int32) * 8
              + lax.broadcasted_iota(jnp.int32, (8, RB), 0))
        gm = jnp.min(om, axis=0, keepdims=True)
        idx_ref[0, 0, :, u * RB:(u + 1) * RB] = jnp.min(
            jnp.where(om == gm, ki, jnp.int32(2**30)), axis=0, keepdims=True
        )


def _nearest(res, cb):
    idx3 = pl.pallas_call(
        _dist_body,
        grid=(KT, RTG),
        in_specs=[
            pl.BlockSpec((U * RB, E_DIM), lambda kt, rtg: (rtg, 0)),
            pl.BlockSpec((KB, E_DIM), lambda kt, rtg: (kt, 0)),
        ],
        out_specs=pl.BlockSpec(
            (1, 1, 1, U * RB), lambda kt, rtg: (kt, rtg, 0, 0)),
        out_shape=jax.ShapeDtypeStruct((KT, RTG, 1, U * RB), jnp.int32),
        scratch_shapes=[
            pltpu.VMEM((KB, E_DIM), jnp.float32),
            pltpu.VMEM((KB, 1), jnp.float32),
            pltpu.VMEM((1, R), jnp.float32),
            pltpu.VMEM((8, R), jnp.float32),
            pltpu.VMEM((8, R), jnp.float32),
        ],
    )(res, cb)
    return idx3[KT - 1].reshape(R)


def _sc_update_body(cb_hbm, idx_hbm, res_hbm, res_out, sq_out,
                    idx_v, q_v, r_v, sq_v, sem):
    wid = lax.axis_index("s") * NC + lax.axis_index("c")
    base = wid * RPW
    pltpu.sync_copy(idx_hbm.at[pl.ds(base, RPW)], idx_v)
    pltpu.async_copy(cb_hbm.at[idx_v], q_v, sem).wait()
    pltpu.sync_copy(res_hbm.at[pl.ds(base, RPW)], r_v)

    def row(i, acc):
        for j in range(E_DIM // 16):
            sl = pl.ds(j * 16, 16)
            r = r_v[i, sl]
            q = q_v[i, sl]
            t = q - r
            q_st = r + t
            r_v[i, sl] = r - q_st
            acc = acc + t * t
        return acc

    acc = lax.fori_loop(0, RPW, row, jnp.zeros((16,), jnp.float32))
    sq_v[...] = acc
    pltpu.sync_copy(r_v, res_out.at[pl.ds(base, RPW)])
    pltpu.sync_copy(sq_v, sq_out.at[wid])


@functools.lru_cache(maxsize=1)
def _build_sc_update():
    return pl.kernel(
        _sc_update_body,
        out_type=(
            jax.ShapeDtypeStruct((R, E_DIM), jnp.float32),
            jax.ShapeDtypeStruct((NW, 16), jnp.float32),
        ),
        mesh=plsc.VectorSubcoreMesh(core_axis_name="c", subcore_axis_name="s",
                                    num_cores=NC, num_subcores=NS),
        scratch_types=[
            pltpu.VMEM((RPW,), jnp.int32),
            pltpu.VMEM((RPW, E_DIM), jnp.float32),
            pltpu.VMEM((RPW, E_DIM), jnp.float32),
            pltpu.VMEM((16,), jnp.float32),
            pltpu.SemaphoreType.DMA,
        ],
    )


DB = 256
DT = B // DB


def _dec_body(res_ref, xe_ref, wdt_ref, bdt_ref, wd_ref, bd_ref,
              wa_ref, ba_ref, sq_ref, xd_ref, rec_ref, al_ref, loss_ref):
    t = pl.program_id(0)
    q = xe_ref[...] - res_ref[...]
    wdt = wdt_ref[...]
    bdt = bdt_ref[...]
    xd_parts = []
    al_parts = []
    for tok in range(T):
        qt = q[:, tok * E_DIM:(tok + 1) * E_DIM]
        xdt = jnp.dot(qt, wdt, preferred_element_type=jnp.float32) + bdt
        xd_parts.append(xdt)
        al_parts.append(
            jnp.dot(xdt, wa_ref[...], preferred_element_type=jnp.float32)
            + ba_ref[...]
        )
    xd = jnp.concatenate(xd_parts, axis=1)
    xd_ref[...] = xd
    rec_ref[...] = (
        jnp.dot(xd, wd_ref[...], preferred_element_type=jnp.float32)
        + bd_ref[...]
    )
    al_ref[...] = jnp.concatenate(al_parts, axis=1)

    @pl.when(t == 0)
    def _():
        loss_ref[0, 0] = (
            (1.0 + BETA) * jnp.sum(sq_ref[...]) / jnp.float32(R * E_DIM)
        )


def _decode(res4, xe, wdt, bdt, wd, bd, wa, ba, sq):
    return pl.pallas_call(
        _dec_body,
        grid=(DT,),
        in_specs=[
            pl.BlockSpec((DB, T * E_DIM), lambda t: (t, 0)),
            pl.BlockSpec((DB, T * E_DIM), lambda t: (t, 0)),
            pl.BlockSpec((E_DIM, E_DIM), lambda t: (0, 0)),
            pl.BlockSpec((1, E_DIM), lambda t: (0, 0)),
            pl.BlockSpec((T * E_DIM, IN_DIM), lambda t: (0, 0)),
            pl.BlockSpec((1, IN_DIM), lambda t: (0, 0)),
            pl.BlockSpec((E_DIM, ALIGN_DIM), lambda t: (0, 0)),
            pl.BlockSpec((1, ALIGN_DIM), lambda t: (0, 0)),
            pl.BlockSpec((N_LAYERS * NW, 16), lambda t: (0, 0)),
        ],
        out_specs=[
            pl.BlockSpec((DB, T * E_DIM), lambda t: (t, 0)),
            pl.BlockSpec((DB, IN_DIM), lambda t: (t, 0)),
            pl.BlockSpec((DB, T * ALIGN_DIM), lambda t: (t, 0)),
            pl.BlockSpec(memory_space=pltpu.SMEM),
        ],
        out_shape=[
            jax.ShapeDtypeStruct((B, T * E_DIM), jnp.float32),
            jax.ShapeDtypeStruct((B, IN_DIM), jnp.float32),
            jax.ShapeDtypeStruct((B, T * ALIGN_DIM), jnp.float32),
            jax.ShapeDtypeStruct((1, 1), jnp.float32),
        ],
    )(res4, xe, wdt, bdt.reshape(1, E_DIM), wd, bd.reshape(1, IN_DIM),
      wa, ba.reshape(1, ALIGN_DIM), sq)


def kernel(video_patches, W_enc, b_enc, cb0, cb1, cb2, cb3,
           W_dec_tok, b_dec_tok, W_dec, b_dec, W_align, b_align):
    xe = _encode(video_patches, W_enc, b_enc)
    res = xe.reshape(R, E_DIM)
    idxs = []
    sqs = []
    for cb in (cb0, cb1, cb2, cb3):
        idx = _nearest(res, cb)
        res, sq = _build_sc_update()(cb, idx, res)
        idxs.append(idx)
        sqs.append(sq)
    sq_all = jnp.concatenate(sqs, axis=0)
    xd, rec, al, loss = _decode(
        res.reshape(B, T * E_DIM), xe, W_dec_tok, b_dec_tok,
        W_dec, b_dec, W_align, b_align, sq_all)
    indices = jnp.stack(idxs, axis=-1).reshape(B, T, N_LAYERS)
    return (
        rec,
        loss.reshape(()),
        indices,
        xe.reshape(B, T, E_DIM),
        xd.reshape(B, T, E_DIM),
        al.reshape(B, T, ALIGN_DIM),
    )
